# baseline (speedup 1.0000x reference)
import functools

import jax
import jax.numpy as jnp
from jax import lax
from jax.experimental import pallas as pl
from jax.experimental.pallas import tpu as pltpu

_C = 256
_IOU_THRESH = 0.5


def _iou_tile(rx1, ry1, rx2, ry2, ra, cx1, cy1, cx2, cy2, ca):
    ix1 = jnp.maximum(rx1, cx1)
    iy1 = jnp.maximum(ry1, cy1)
    ix2 = jnp.minimum(rx2, cx2)
    iy2 = jnp.minimum(ry2, cy2)
    inter = jnp.clip(ix2 - ix1, 0.0) * jnp.clip(iy2 - iy1, 0.0)
    return inter / (ra + ca - inter + 1e-9)


def _nms_body(nc, x1_ref, y1_ref, x2_ref, y2_ref, keep_ref, sbuf_ref):
    C = _C
    sbuf_ref[...] = jnp.zeros(sbuf_ref.shape, jnp.float32)

    ii = lax.broadcasted_iota(jnp.int32, (C, C), 0)
    jj = lax.broadcasted_iota(jnp.int32, (C, C), 1)
    upper = ii < jj
    uf = jnp.where(upper, 1.0, 0.0)

    def chunk_step(c, cnt):
        cx1 = x1_ref[pl.ds(c, 1), :]
        cy1 = y1_ref[pl.ds(c, 1), :]
        cx2 = x2_ref[pl.ds(c, 1), :]
        cy2 = y2_ref[pl.ds(c, 1), :]
        ca = (cx2 - cx1) * (cy2 - cy1)

        ntiles = (cnt + (C - 1)) // C

        def tstep(t, smax):
            sb = sbuf_ref[pl.ds(t * C, C), :]
            iou = _iou_tile(sb[:, 0:1], sb[:, 1:2], sb[:, 2:3], sb[:, 3:4],
                            sb[:, 4:5], cx1, cy1, cx2, cy2, ca)
            return jnp.maximum(smax, jnp.max(iou, axis=0, keepdims=True))

        smax = lax.fori_loop(0, ntiles, tstep, jnp.zeros((1, C), jnp.float32))
        k0 = jnp.where(smax >= _IOU_THRESH, 0.0, 1.0)

        rx1 = cx1.reshape(C, 1)
        ry1 = cy1.reshape(C, 1)
        rx2 = cx2.reshape(C, 1)
        ry2 = cy2.reshape(C, 1)
        ra = ca.reshape(C, 1)
        iou_d = _iou_tile(rx1, ry1, rx2, ry2, ra, cx1, cy1, cx2, cy2, ca)
        mf = jnp.where((iou_d >= _IOU_THRESH) & upper, 1.0, 0.0)

        def fix_cond(carry):
            return carry[1]

        def fix_body(carry):
            k, _ = carry
            s = jnp.max(mf * k.reshape(C, 1), axis=0, keepdims=True)
            kn = k0 * (1.0 - s)
            return kn, jnp.any(kn != k)

        kf, _ = lax.while_loop(fix_cond, fix_body, (k0, True))
        keep_ref[pl.ds(c, 1), :] = kf

        kcol = kf.reshape(C, 1)
        pos = jnp.dot(kf, uf, preferred_element_type=jnp.float32,
                      precision=lax.Precision.HIGHEST)
        perm = jnp.where((jj == pos.reshape(C, 1).astype(jnp.int32)) & (kcol > 0.0),
                         1.0, 0.0)
        data = jnp.concatenate(
            [rx1, ry1, rx2, ry2, ra, jnp.zeros((C, 3), jnp.float32)], axis=1)
        packed = lax.dot_general(perm, data, (((0,), (0,)), ((), ())),
                                 preferred_element_type=jnp.float32,
                                 precision=lax.Precision.HIGHEST)
        sbuf_ref[pl.ds(cnt, C), :] = packed
        return cnt + jnp.sum(kf).astype(jnp.int32)

    lax.fori_loop(0, nc, chunk_step, jnp.int32(0))


@jax.jit
def kernel(boxes, scores):
    n = boxes.shape[0]
    nc = (n + _C - 1) // _C
    npad = nc * _C

    order = jnp.argsort(-scores)
    b = jnp.take(boxes, order, axis=0)
    s = jnp.take(scores, order, axis=0)

    bp = jnp.pad(b, ((0, npad - n), (0, 0)))
    x1 = bp[:, 0].reshape(nc, _C)
    y1 = bp[:, 1].reshape(nc, _C)
    x2 = bp[:, 2].reshape(nc, _C)
    y2 = bp[:, 3].reshape(nc, _C)

    keep = pl.pallas_call(
        functools.partial(_nms_body, nc),
        out_shape=jax.ShapeDtypeStruct((nc, _C), jnp.float32),
        scratch_shapes=[pltpu.VMEM((npad + _C, 8), jnp.float32)],
    )(x1, y1, x2, y2)

    keepf = keep.reshape(npad)[:n]
    return jnp.concatenate([b * keepf[:, None], (s * keepf)[:, None]], axis=1)


# speedup vs baseline: 1.2185x; 1.2185x over previous
import functools

import jax
import jax.numpy as jnp
from jax import lax
from jax.experimental import pallas as pl
from jax.experimental.pallas import tpu as pltpu

_C = 256
_R = 64
_IOU_THRESH = 0.5


def _iou_tile(rx1, ry1, rx2, ry2, ra, cx1, cy1, cx2, cy2, ca):
    ix1 = jnp.maximum(rx1, cx1)
    iy1 = jnp.maximum(ry1, cy1)
    ix2 = jnp.minimum(rx2, cx2)
    iy2 = jnp.minimum(ry2, cy2)
    inter = jnp.clip(ix2 - ix1, 0.0) * jnp.clip(iy2 - iy1, 0.0)
    return inter / (ra + ca - inter + 1e-9)


def _nms_body(nc, x1_ref, y1_ref, x2_ref, y2_ref, keep_ref, area_ref):
    C = _C
    R = _R
    keep_ref[...] = jnp.ones((nc, C), jnp.float32)
    area_ref[...] = (x2_ref[...] - x1_ref[...]) * (y2_ref[...] - y1_ref[...])

    ii = lax.broadcasted_iota(jnp.int32, (C, C), 0)
    jj = lax.broadcasted_iota(jnp.int32, (C, C), 1)
    upper = ii < jj

    def chunk_step(c, _):
        rx1r = x1_ref[pl.ds(c, 1), :]
        ry1r = y1_ref[pl.ds(c, 1), :]
        rx2r = x2_ref[pl.ds(c, 1), :]
        ry2r = y2_ref[pl.ds(c, 1), :]
        rar = area_ref[pl.ds(c, 1), :]
        rx1 = rx1r.reshape(C, 1)
        ry1 = ry1r.reshape(C, 1)
        rx2 = rx2r.reshape(C, 1)
        ry2 = ry2r.reshape(C, 1)
        ra = rar.reshape(C, 1)

        iou_d = _iou_tile(rx1, ry1, rx2, ry2, ra, rx1r, ry1r, rx2r, ry2r, rar)
        mf = jnp.where((iou_d >= _IOU_THRESH) & upper, 1.0, 0.0)
        k0 = keep_ref[pl.ds(c, 1), :]

        def fix_cond(carry):
            return carry[1]

        def fix_body(carry):
            k, _ = carry
            s = jnp.max(mf * k.reshape(C, 1), axis=0, keepdims=True)
            kn = k0 * (1.0 - s)
            return kn, jnp.any(kn != k)

        kf, _ = lax.while_loop(fix_cond, fix_body, (k0, True))
        keep_ref[pl.ds(c, 1), :] = kf

        kcol = kf.reshape(C, 1)
        mx1 = rx1 * kcol
        my1 = ry1 * kcol
        mx2 = rx2 * kcol
        my2 = ry2 * kcol

        def jstep(j, _):
            cx1 = x1_ref[pl.ds(j, 1), :]
            cy1 = y1_ref[pl.ds(j, 1), :]
            cx2 = x2_ref[pl.ds(j, 1), :]
            cy2 = y2_ref[pl.ds(j, 1), :]
            ca = area_ref[pl.ds(j, 1), :]
            smax = jnp.zeros((1, C), jnp.float32)
            for r in range(0, C, R):
                iou = _iou_tile(mx1[r:r + R], my1[r:r + R],
                                mx2[r:r + R], my2[r:r + R], ra[r:r + R],
                                cx1, cy1, cx2, cy2, ca)
                smax = jnp.maximum(smax, jnp.max(iou, axis=0, keepdims=True))
            supp = jnp.where(smax >= _IOU_THRESH, 1.0, 0.0)
            keep_ref[pl.ds(j, 1), :] = keep_ref[pl.ds(j, 1), :] * (1.0 - supp)
            return 0

        lax.fori_loop(c + 1, nc, jstep, 0)
        return 0

    lax.fori_loop(0, nc, chunk_step, 0)


@jax.jit
def kernel(boxes, scores):
    n = boxes.shape[0]
    nc = (n + _C - 1) // _C
    npad = nc * _C

    order = jnp.argsort(-scores)
    b = jnp.take(boxes, order, axis=0)
    s = jnp.take(scores, order, axis=0)

    bp = jnp.pad(b, ((0, npad - n), (0, 0)))
    x1 = bp[:, 0].reshape(nc, _C)
    y1 = bp[:, 1].reshape(nc, _C)
    x2 = bp[:, 2].reshape(nc, _C)
    y2 = bp[:, 3].reshape(nc, _C)

    keep = pl.pallas_call(
        functools.partial(_nms_body, nc),
        out_shape=jax.ShapeDtypeStruct((nc, _C), jnp.float32),
        scratch_shapes=[pltpu.VMEM((nc, _C), jnp.float32)],
    )(x1, y1, x2, y2)

    keepf = keep.reshape(npad)[:n]
    return jnp.concatenate([b * keepf[:, None], (s * keepf)[:, None]], axis=1)


# speedup vs baseline: 1.2484x; 1.0245x over previous
import functools

import jax
import jax.numpy as jnp
from jax import lax
from jax.experimental import pallas as pl
from jax.experimental.pallas import tpu as pltpu

_C = 256
_R = 128
_IOU_THRESH = 0.5


def _iou_tile(rx1, ry1, rx2, ry2, ra, cx1, cy1, cx2, cy2, ca):
    ix1 = jnp.maximum(rx1, cx1)
    iy1 = jnp.maximum(ry1, cy1)
    ix2 = jnp.minimum(rx2, cx2)
    iy2 = jnp.minimum(ry2, cy2)
    inter = jnp.clip(ix2 - ix1, 0.0) * jnp.clip(iy2 - iy1, 0.0)
    return inter / (ra + ca - inter + 1e-9)


def _nms_body(nc, x1_ref, y1_ref, x2_ref, y2_ref, keep_ref, area_ref):
    C = _C
    R = _R
    keep_ref[...] = jnp.ones((nc, C), jnp.float32)
    area_ref[...] = (x2_ref[...] - x1_ref[...]) * (y2_ref[...] - y1_ref[...])

    ii = lax.broadcasted_iota(jnp.int32, (C, C), 0)
    jj = lax.broadcasted_iota(jnp.int32, (C, C), 1)
    upper = ii < jj

    def chunk_step(c, _):
        rx1r = x1_ref[pl.ds(c, 1), :]
        ry1r = y1_ref[pl.ds(c, 1), :]
        rx2r = x2_ref[pl.ds(c, 1), :]
        ry2r = y2_ref[pl.ds(c, 1), :]
        rar = area_ref[pl.ds(c, 1), :]
        rx1 = rx1r.reshape(C, 1)
        ry1 = ry1r.reshape(C, 1)
        rx2 = rx2r.reshape(C, 1)
        ry2 = ry2r.reshape(C, 1)
        ra = rar.reshape(C, 1)

        iou_d = _iou_tile(rx1, ry1, rx2, ry2, ra, rx1r, ry1r, rx2r, ry2r, rar)
        mf = jnp.where((iou_d >= _IOU_THRESH) & upper, 1.0, 0.0)
        k0 = keep_ref[pl.ds(c, 1), :]

        def fix_cond(carry):
            return carry[1]

        def fix_body(carry):
            k, _ = carry
            s = jnp.max(mf * k.reshape(C, 1), axis=0, keepdims=True)
            kn = k0 * (1.0 - s)
            return kn, jnp.any(kn != k)

        kf, _ = lax.while_loop(fix_cond, fix_body, (k0, True))
        keep_ref[pl.ds(c, 1), :] = kf

        kcol = kf.reshape(C, 1)
        mx1 = rx1 * kcol
        my1 = ry1 * kcol
        mx2 = rx2 * kcol
        my2 = ry2 * kcol

        def jstep(j, _):
            cx1 = x1_ref[pl.ds(j, 1), :]
            cy1 = y1_ref[pl.ds(j, 1), :]
            cx2 = x2_ref[pl.ds(j, 1), :]
            cy2 = y2_ref[pl.ds(j, 1), :]
            ca = area_ref[pl.ds(j, 1), :]
            smax = jnp.zeros((1, C), jnp.float32)
            for r in range(0, C, R):
                iou = _iou_tile(mx1[r:r + R], my1[r:r + R],
                                mx2[r:r + R], my2[r:r + R], ra[r:r + R],
                                cx1, cy1, cx2, cy2, ca)
                smax = jnp.maximum(smax, jnp.max(iou, axis=0, keepdims=True))
            supp = jnp.where(smax >= _IOU_THRESH, 1.0, 0.0)
            keep_ref[pl.ds(j, 1), :] = keep_ref[pl.ds(j, 1), :] * (1.0 - supp)
            return 0

        lax.fori_loop(c + 1, nc, jstep, 0)
        return 0

    lax.fori_loop(0, nc, chunk_step, 0)


@jax.jit
def kernel(boxes, scores):
    n = boxes.shape[0]
    nc = (n + _C - 1) // _C
    npad = nc * _C

    order = jnp.argsort(-scores)
    b = jnp.take(boxes, order, axis=0)
    s = jnp.take(scores, order, axis=0)

    bp = jnp.pad(b, ((0, npad - n), (0, 0)))
    x1 = bp[:, 0].reshape(nc, _C)
    y1 = bp[:, 1].reshape(nc, _C)
    x2 = bp[:, 2].reshape(nc, _C)
    y2 = bp[:, 3].reshape(nc, _C)

    keep = pl.pallas_call(
        functools.partial(_nms_body, nc),
        out_shape=jax.ShapeDtypeStruct((nc, _C), jnp.float32),
        scratch_shapes=[pltpu.VMEM((nc, _C), jnp.float32)],
    )(x1, y1, x2, y2)

    keepf = keep.reshape(npad)[:n]
    return jnp.concatenate([b * keepf[:, None], (s * keepf)[:, None]], axis=1)
